# baseline (speedup 1.0000x reference)
import functools

import jax
import jax.numpy as jnp
from jax.experimental import pallas as pl
from jax.experimental.pallas import tpu as pltpu

_DR = 8


def _topk_kern(x_ref, w_ref, b_ref, out_ref, acc, *, v_total, dr):
    j = pl.program_id(0)
    nj = pl.num_programs(0)

    part = x_ref[...] * w_ref[...]

    @pl.when(j == 0)
    def _init():
        acc[...] = part

    @pl.when(j > 0)
    def _accum():
        acc[...] += part

    @pl.when(j == nj - 1)
    def _emit():
        logits = jnp.sum(acc[...], axis=0, keepdims=True) + b_ref[...]
        m = jnp.max(logits)
        col = jax.lax.broadcasted_iota(jnp.int32, logits.shape, 1)
        out_ref[0] = jnp.min(
            jnp.where(logits == m, col, jnp.iinfo(jnp.int32).max)
        )


def kernel(x, W, b):
    d, v = W.shape
    dr = _DR if d % _DR == 0 else 1
    nj = d // dr
    x0 = x[0:1].reshape(d, 1)
    b2 = b.reshape(1, v)
    out = pl.pallas_call(
        functools.partial(_topk_kern, v_total=v, dr=dr),
        grid=(nj,),
        in_specs=[
            pl.BlockSpec((dr, 1), lambda j: (j, 0)),
            pl.BlockSpec((dr, v), lambda j: (j, 0)),
            pl.BlockSpec((1, v), lambda j: (0, 0)),
        ],
        out_specs=pl.BlockSpec(memory_space=pltpu.SMEM),
        out_shape=jax.ShapeDtypeStruct((1,), jnp.int32),
        scratch_shapes=[
            pltpu.VMEM((dr, v), jnp.float32),
        ],
        compiler_params=pltpu.CompilerParams(
            dimension_semantics=("arbitrary",),
        ),
    )(x0, W, b2)
    return out


# speedup vs baseline: 1.0980x; 1.0980x over previous
import functools

import jax
import jax.numpy as jnp
from jax.experimental import pallas as pl
from jax.experimental.pallas import tpu as pltpu

_CR = 8
_NBUF = 8


def _topk_kern(x_ref, w_hbm, b_ref, out_ref, buf, acc, sem, *, nchunks, cr, nbuf):
    def start_copy(chunk, slot):
        pltpu.make_async_copy(
            w_hbm.at[pl.ds(chunk * cr, cr), :], buf.at[slot], sem.at[slot]
        ).start()

    def wait_copy(chunk, slot):
        pltpu.make_async_copy(
            w_hbm.at[pl.ds(chunk * cr, cr), :], buf.at[slot], sem.at[slot]
        ).wait()

    for i in range(min(nbuf, nchunks)):
        start_copy(i, i)

    acc[...] = jnp.zeros_like(acc)

    def body(j, carry):
        slot = jax.lax.rem(j, nbuf)
        wait_copy(j, slot)
        xk = x_ref[pl.ds(j * cr, cr), :]
        acc[...] += xk * buf[slot]

        @pl.when(j + nbuf < nchunks)
        def _():
            start_copy(j + nbuf, slot)

        return carry

    jax.lax.fori_loop(0, nchunks, body, 0)

    logits = jnp.sum(acc[...], axis=0, keepdims=True) + b_ref[...]
    m = jnp.max(logits)
    col = jax.lax.broadcasted_iota(jnp.int32, logits.shape, 1)
    out_ref[0] = jnp.min(jnp.where(logits == m, col, jnp.iinfo(jnp.int32).max))


def kernel(x, W, b):
    d, v = W.shape
    cr = _CR if d % _CR == 0 else d
    nchunks = d // cr
    nbuf = min(_NBUF, nchunks)
    x0 = x[0:1].reshape(d, 1)
    b2 = b.reshape(1, v)
    out = pl.pallas_call(
        functools.partial(_topk_kern, nchunks=nchunks, cr=cr, nbuf=nbuf),
        in_specs=[
            pl.BlockSpec(memory_space=pltpu.VMEM),
            pl.BlockSpec(memory_space=pl.ANY),
            pl.BlockSpec(memory_space=pltpu.VMEM),
        ],
        out_specs=pl.BlockSpec(memory_space=pltpu.SMEM),
        out_shape=jax.ShapeDtypeStruct((1,), jnp.int32),
        scratch_shapes=[
            pltpu.VMEM((nbuf, cr, v), jnp.float32),
            pltpu.VMEM((cr, v), jnp.float32),
            pltpu.SemaphoreType.DMA((nbuf,)),
        ],
    )(x0, W, b2)
    return out


# speedup vs baseline: 4.3008x; 3.9169x over previous
import functools

import jax
import jax.numpy as jnp
from jax.experimental import pallas as pl
from jax.experimental.pallas import tpu as pltpu

_TVS = 1024


def _topk_kern(x_ref, wt_ref, b_ref, out_ref, best_val, best_idx, *, v_total, tvs):
    j = pl.program_id(0)
    nj = pl.num_programs(0)

    @pl.when(j == 0)
    def _init():
        best_val[0] = -jnp.inf
        best_idx[0] = 0

    rs = jnp.sum(wt_ref[...] * x_ref[...], axis=1, keepdims=True)
    score = rs + jnp.transpose(b_ref[...], (1, 0))
    ri = j * tvs + jax.lax.broadcasted_iota(jnp.int32, score.shape, 0)
    score = jnp.where(ri < v_total, score, -jnp.inf)
    m = jnp.max(score)
    li = jnp.min(jnp.where(score == m, ri, jnp.iinfo(jnp.int32).max))

    @pl.when(m > best_val[0])
    def _update():
        best_val[0] = m
        best_idx[0] = li

    @pl.when(j == nj - 1)
    def _emit():
        out_ref[0] = best_idx[0]


def kernel(x, W, b):
    d, v = W.shape
    tvs = min(_TVS, v)
    nj = pl.cdiv(v, tvs)
    wt = W.T
    x0 = x[0:1]
    b2 = b.reshape(1, v)
    out = pl.pallas_call(
        functools.partial(_topk_kern, v_total=v, tvs=tvs),
        grid=(nj,),
        in_specs=[
            pl.BlockSpec((1, d), lambda j: (0, 0)),
            pl.BlockSpec((tvs, d), lambda j: (j, 0)),
            pl.BlockSpec((1, tvs), lambda j: (0, j)),
        ],
        out_specs=pl.BlockSpec(memory_space=pltpu.SMEM),
        out_shape=jax.ShapeDtypeStruct((1,), jnp.int32),
        scratch_shapes=[
            pltpu.SMEM((1,), jnp.float32),
            pltpu.SMEM((1,), jnp.int32),
        ],
        compiler_params=pltpu.CompilerParams(
            dimension_semantics=("arbitrary",),
        ),
    )(x0, wt, b2)
    return out
